# baseline (speedup 1.0000x reference)
import jax
import jax.numpy as jnp
from jax import lax
from jax.experimental import pallas as pl
from jax.experimental.pallas import tpu as pltpu
from jax.experimental.pallas import tpu_sc as plsc

_IGNORE_INDEX = 10

_NC = 2
_NS = 16
_NW = _NC * _NS
_L = 16


_ZLEN = 16000


def _make_fill_kernel(N, C):
    total = N * C
    region = total // _NW
    n_zero_dmas = region // _ZLEN
    assert total % _NW == 0 and region % _ZLEN == 0

    mesh = plsc.VectorSubcoreMesh(core_axis_name="c", subcore_axis_name="s")

    @pl.kernel(
        mesh=mesh,
        out_type=(),
        scratch_types=[
            pltpu.VMEM((_ZLEN,), jnp.float32),
            pltpu.SemaphoreType.DMA,
        ],
    )
    def kern(buf_hbm, zbuf, zsem):
        wid = lax.axis_index("s") * _NC + lax.axis_index("c")
        flat_base = wid * region
        zeros16 = jnp.zeros((_L,), jnp.float32)

        def zero_step(i, carry):
            base = i * (8 * _L)
            for k in range(8):
                zbuf[pl.ds(base + k * _L, _L)] = zeros16
            return carry

        lax.fori_loop(0, _ZLEN // (8 * _L), zero_step, 0)

        fills = []
        for j in range(n_zero_dmas):
            fills.append(
                pltpu.async_copy(
                    zbuf, buf_hbm.at[pl.ds(flat_base + j * _ZLEN, _ZLEN)],
                    zsem))
        for f in fills:
            f.wait()

    return kern


def _make_scatter_kernel(N, C):
    rows_per_w = N // _NW
    assert N % _NW == 0 and rows_per_w % 128 == 0
    idx_rows = rows_per_w // 128

    mesh = plsc.VectorSubcoreMesh(core_axis_name="c", subcore_axis_name="s")

    n_vec = rows_per_w // _L

    @pl.kernel(
        mesh=mesh,
        out_type=(),
        scratch_types=[
            pltpu.VMEM((rows_per_w,), jnp.int32),
            pltpu.VMEM((rows_per_w,), jnp.float32),
            pltpu.VMEM((idx_rows, 128), jnp.int32),
            pltpu.VMEM((idx_rows, 128), jnp.float32),
            pltpu.SemaphoreType.DMA,
        ],
    )
    def kern(buf_hbm, tgt_hbm, grd_hbm, tgt_v, grd_v, idx_v, val_v, ssem):
        wid = lax.axis_index("s") * _NC + lax.axis_index("c")
        row_base = wid * rows_per_w
        pltpu.sync_copy(tgt_hbm.at[pl.ds(row_base, rows_per_w)], tgt_v)
        pltpu.sync_copy(grd_hbm.at[pl.ds(row_base, rows_per_w)], grd_v)

        lane = lax.iota(jnp.int32, _L)
        for i in range(n_vec):
            t = tgt_v[pl.ds(i * _L, _L)]
            g = grd_v[pl.ds(i * _L, _L)]
            valid = t != _IGNORE_INDEX
            vals = jnp.where(valid, -g, jnp.zeros((_L,), jnp.float32))
            rows = (row_base + i * _L) + lane
            flat = t * N + rows
            r, col = i // 8, (i % 8) * _L
            idx_v[r, pl.ds(col, _L)] = flat
            val_v[r, pl.ds(col, _L)] = vals

        scats = []
        for r in range(idx_rows):
            scats.append(
                pltpu.async_copy(val_v.at[r], buf_hbm.at[idx_v.at[r]], ssem))
        for s in scats:
            s.wait()

    return kern


def kernel(grad_output, input, target, total_weight):
    N, C = input.shape
    t = target.astype(jnp.int32)
    g = grad_output.astype(jnp.float32)
    buf = jax.new_ref(jnp.empty((N * C,), jnp.float32))
    _make_fill_kernel(N, C)(buf)
    _make_scatter_kernel(N, C)(buf, t, g)
    return buf[...].reshape(C, N).T


# speedup vs baseline: 2.4931x; 2.4931x over previous
import jax
import jax.numpy as jnp
from jax import lax
from jax.experimental import pallas as pl
from jax.experimental.pallas import tpu as pltpu
from jax.experimental.pallas import tpu_sc as plsc

_IGNORE_INDEX = 10

_NC = 2
_NS = 16
_NW = _NC * _NS
_L = 16

_CB = 8
_NQ = 4
_MAXK = 16


def _make_sc_kernel(N, C):
    NB = N // _NQ
    n_tile_rows = C // _CB
    assert C % _CB == 0 and N % (_NQ * 128) == 0 and NB % _L == 0
    n_cg = NB // _L

    mesh = plsc.VectorSubcoreMesh(core_axis_name="c", subcore_axis_name="s")

    @pl.kernel(
        mesh=mesh,
        compiler_params=pltpu.CompilerParams(use_tc_tiling_on_sc=True),
        out_type=jax.ShapeDtypeStruct((C, N), jnp.float32),
        scratch_types=[
            pltpu.VMEM((_CB, NB), jnp.float32),
            pltpu.VMEM((_CB, NB), jnp.float32),
            pltpu.VMEM((NB,), jnp.int32),
            pltpu.VMEM((NB,), jnp.float32),
            pltpu.SemaphoreType.DMA,
            pltpu.SemaphoreType.DMA,
        ],
    )
    def kern(tgt_hbm, grd_hbm, out_hbm, buf0, buf1, tgt_v, grd_v,
             sem0, sem1):
        worker = lax.axis_index("s") * _NC + lax.axis_index("c")
        q = worker % _NQ
        jgrp = worker // _NQ
        col_base = q * NB

        pltpu.sync_copy(tgt_hbm.at[pl.ds(col_base, NB)], tgt_v)
        pltpu.sync_copy(grd_hbm.at[pl.ds(col_base, NB)], grd_v)

        zeros16 = jnp.zeros((_L,), jnp.float32)
        bufs = (buf0, buf1)
        sems = (sem0, sem1)

        def build_and_send(k, buf, sem):
            tile_row = jgrp + 8 * k
            j0 = tile_row * _CB

            @pl.when(tile_row < n_tile_rows)
            def _():
                @pl.when(k >= 2)
                def _():
                    pltpu.make_async_copy(
                        buf,
                        out_hbm.at[pl.ds((jgrp + 8 * (k - 2)) * _CB, _CB),
                                   pl.ds(col_base, NB)],
                        sem).wait()

                def cg_step(cg, carry):
                    t = tgt_v[pl.ds(cg * _L, _L)]
                    g = grd_v[pl.ds(cg * _L, _L)]
                    for s in range(_CB):
                        buf[s, pl.ds(cg * _L, _L)] = jnp.where(
                            t == j0 + s, g, zeros16)
                    return carry

                lax.fori_loop(0, n_cg, cg_step, 0)
                pltpu.async_copy(
                    buf,
                    out_hbm.at[pl.ds(j0, _CB), pl.ds(col_base, NB)],
                    sem)

        def step(k, carry):
            @pl.when(k % 2 == 0)
            def _():
                build_and_send(k, buf0, sem0)

            @pl.when(k % 2 == 1)
            def _():
                build_and_send(k, buf1, sem1)

            return carry

        lax.fori_loop(0, _MAXK, step, 0)

        def drain(k, buf, sem):
            tile_row = jgrp + 8 * k

            @pl.when(tile_row < n_tile_rows)
            def _():
                pltpu.make_async_copy(
                    buf,
                    out_hbm.at[pl.ds(tile_row * _CB, _CB),
                               pl.ds(col_base, NB)],
                    sem).wait()

        drain(_MAXK - 2, buf0, sem0)
        drain(_MAXK - 1, buf1, sem1)

    return kern


def kernel(grad_output, input, target, total_weight):
    N, C = input.shape
    t = target.astype(jnp.int32)
    g = grad_output.astype(jnp.float32)
    g2 = jnp.where(t != _IGNORE_INDEX, -g, jnp.zeros_like(g))
    out_t = _make_sc_kernel(N, C)(t, g2)
    return out_t.T


# speedup vs baseline: 2.5645x; 1.0286x over previous
import jax
import jax.numpy as jnp
from jax import lax
from jax.experimental import pallas as pl
from jax.experimental.pallas import tpu as pltpu
from jax.experimental.pallas import tpu_sc as plsc

_IGNORE_INDEX = 10

_NC = 2
_NS = 16
_NW = _NC * _NS
_L = 16

_CB = 8
_NQ = 4
_MAXK = 16


def _make_sc_kernel(N, C):
    NB = N // _NQ
    n_tile_rows = C // _CB
    assert C % _CB == 0 and N % (_NQ * 128) == 0 and NB % _L == 0
    n_cg = NB // _L

    mesh = plsc.VectorSubcoreMesh(core_axis_name="c", subcore_axis_name="s")

    @pl.kernel(
        mesh=mesh,
        compiler_params=pltpu.CompilerParams(use_tc_tiling_on_sc=True),
        out_type=jax.ShapeDtypeStruct((C, N), jnp.float32),
        scratch_types=[
            pltpu.VMEM((_CB, NB), jnp.float32),
            pltpu.VMEM((_CB, NB), jnp.float32),
            pltpu.VMEM((NB,), jnp.int32),
            pltpu.VMEM((NB,), jnp.float32),
            pltpu.SemaphoreType.DMA,
            pltpu.SemaphoreType.DMA,
        ],
    )
    def kern(tgt_hbm, grd_hbm, out_hbm, buf0, buf1, tgt_v, grd_v,
             sem0, sem1):
        worker = lax.axis_index("s") * _NC + lax.axis_index("c")
        q = worker % _NQ
        jgrp = worker // _NQ
        col_base = q * NB

        pltpu.sync_copy(tgt_hbm.at[pl.ds(col_base, NB)], tgt_v)
        pltpu.sync_copy(grd_hbm.at[pl.ds(col_base, NB)], grd_v)

        zeros16 = jnp.zeros((_L,), jnp.float32)
        bufs = (buf0, buf1)
        sems = (sem0, sem1)

        def build_and_send(k, buf, sem):
            tile_row = jgrp + 8 * k
            j0 = tile_row * _CB

            @pl.when(tile_row < n_tile_rows)
            def _():
                @pl.when(k >= 2)
                def _():
                    pltpu.make_async_copy(
                        buf,
                        out_hbm.at[pl.ds((jgrp + 8 * (k - 2)) * _CB, _CB),
                                   pl.ds(col_base, NB)],
                        sem).wait()

                def cg_step(cg, carry):
                    base = cg * (4 * _L)
                    for u in range(4):
                        off = base + u * _L
                        t = tgt_v[pl.ds(off, _L)]
                        g = grd_v[pl.ds(off, _L)]
                        for s in range(_CB):
                            buf[s, pl.ds(off, _L)] = jnp.where(
                                t == j0 + s, g, zeros16)
                    return carry

                lax.fori_loop(0, n_cg // 4, cg_step, 0)
                pltpu.async_copy(
                    buf,
                    out_hbm.at[pl.ds(j0, _CB), pl.ds(col_base, NB)],
                    sem)

        def step(k, carry):
            @pl.when(k % 2 == 0)
            def _():
                build_and_send(k, buf0, sem0)

            @pl.when(k % 2 == 1)
            def _():
                build_and_send(k, buf1, sem1)

            return carry

        lax.fori_loop(0, _MAXK, step, 0)

        def drain(k, buf, sem):
            tile_row = jgrp + 8 * k

            @pl.when(tile_row < n_tile_rows)
            def _():
                pltpu.make_async_copy(
                    buf,
                    out_hbm.at[pl.ds(tile_row * _CB, _CB),
                               pl.ds(col_base, NB)],
                    sem).wait()

        drain(_MAXK - 2, buf0, sem0)
        drain(_MAXK - 1, buf1, sem1)

    return kern


def kernel(grad_output, input, target, total_weight):
    N, C = input.shape
    t = target.astype(jnp.int32)
    g = grad_output.astype(jnp.float32)
    g2 = jnp.where(t != _IGNORE_INDEX, -g, jnp.zeros_like(g))
    out_t = _make_sc_kernel(N, C)(t, g2)
    return out_t.T
